# Initial kernel scaffold; baseline (speedup 1.0000x reference)
import jax
import jax.numpy as jnp
from jax.experimental import pallas as pl


def kernel(inputs, W_e1, b_e1, W_e2, b_e2, W_e3, b_e3, codebook, W3, b3, W4, b4, W5, b5):
    raise NotImplementedError("write your pallas kernel here")


# speedup vs baseline: 1.0752x; 1.0752x over previous
import functools

import jax
import jax.numpy as jnp
from jax import lax
from jax.experimental import pallas as pl
from jax.experimental.pallas import tpu as pltpu
from jax.experimental.pallas import tpu_sc as plsc

B, IN, H, D, K, S = 16384, 256, 512, 32, 8192, 256

BB = 256
BD = 512

_NC, _NS = 2, 16
_NW = _NC * _NS
_BPW = B // _NW


def _encode_argmin_body(x_ref, we1_ref, be1_ref, we2_ref, be2_ref,
                        we3_ref, be3_ref, cbt_ref, ze_ref, am_ref):
    x = x_ref[...]
    h = jnp.maximum(
        jnp.dot(x, we1_ref[...], preferred_element_type=jnp.float32)
        + be1_ref[...], 0.0)
    h = jnp.maximum(
        jnp.dot(h, we2_ref[...], preferred_element_type=jnp.float32)
        + be2_ref[...], 0.0)
    z = (jnp.dot(h, we3_ref[...], preferred_element_type=jnp.float32)
         + be3_ref[...])
    ze_ref[...] = z
    cbt = cbt_ref[...]
    d2 = (jnp.sum(z * z, axis=1, keepdims=True)
          - 2.0 * jnp.dot(z, cbt, preferred_element_type=jnp.float32)
          + jnp.sum(cbt * cbt, axis=0)[None, :])
    minv = jnp.min(d2, axis=1, keepdims=True)
    iota = lax.broadcasted_iota(jnp.int32, d2.shape, 1)
    am = jnp.min(jnp.where(d2 == minv, iota, K), axis=1)
    am_ref[...] = am.astype(jnp.int32)[:, None]


def _decode_body(zq_ref, w3_ref, b3_ref, w4_ref, b4_ref, w5_ref, b5_ref,
                 out_ref):
    h = jnp.maximum(
        jnp.dot(zq_ref[...], w3_ref[...], preferred_element_type=jnp.float32)
        + b3_ref[...], 0.0)
    h = jnp.maximum(
        jnp.dot(h, w4_ref[...], preferred_element_type=jnp.float32)
        + b4_ref[...], 0.0)
    out_ref[...] = (
        jnp.dot(h, w5_ref[...], preferred_element_type=jnp.float32)
        + b5_ref[...])


def _sc_gather(codebook, idx):
    mesh = plsc.VectorSubcoreMesh(core_axis_name="c", subcore_axis_name="s")

    @functools.partial(
        pl.kernel, mesh=mesh,
        compiler_params=pltpu.CompilerParams(use_tc_tiling_on_sc=False),
        out_type=jax.ShapeDtypeStruct((B, D), jnp.float32),
        scratch_types=[
            pltpu.VMEM((_BPW,), jnp.int32),
            pltpu.VMEM((_BPW, D), jnp.float32),
            pltpu.SemaphoreType.DMA,
        ],
    )
    def gather_k(table_hbm, idx_hbm, out_hbm, idx_v, rows_v, sem):
        wid = lax.axis_index("s") * _NC + lax.axis_index("c")
        base = wid * _BPW
        pltpu.sync_copy(idx_hbm.at[pl.ds(base, _BPW)], idx_v)
        pltpu.async_copy(table_hbm.at[idx_v], rows_v, sem).wait()
        pltpu.sync_copy(rows_v, out_hbm.at[pl.ds(base, _BPW)])

    return gather_k(codebook, idx)


def kernel(inputs, W_e1, b_e1, W_e2, b_e2, W_e3, b_e3, codebook,
           W3, b3, W4, b4, W5, b5):
    be1 = b_e1.reshape(1, H)
    be2 = b_e2.reshape(1, H)
    be3 = b_e3.reshape(1, D)
    b3r = b3.reshape(1, H)
    b4r = b4.reshape(1, H)
    b5r = b5.reshape(1, S)
    cbt = codebook.T

    z_e, am2 = pl.pallas_call(
        _encode_argmin_body,
        grid=(B // BB,),
        in_specs=[
            pl.BlockSpec((BB, IN), lambda i: (i, 0)),
            pl.BlockSpec((IN, H), lambda i: (0, 0)),
            pl.BlockSpec((1, H), lambda i: (0, 0)),
            pl.BlockSpec((H, H), lambda i: (0, 0)),
            pl.BlockSpec((1, H), lambda i: (0, 0)),
            pl.BlockSpec((H, D), lambda i: (0, 0)),
            pl.BlockSpec((1, D), lambda i: (0, 0)),
            pl.BlockSpec((D, K), lambda i: (0, 0)),
        ],
        out_specs=[
            pl.BlockSpec((BB, D), lambda i: (i, 0)),
            pl.BlockSpec((BB, 1), lambda i: (i, 0)),
        ],
        out_shape=[
            jax.ShapeDtypeStruct((B, D), jnp.float32),
            jax.ShapeDtypeStruct((B, 1), jnp.int32),
        ],
    )(inputs, W_e1, be1, W_e2, be2, W_e3, be3, cbt)

    argmin = am2.reshape(B)
    emb = _sc_gather(codebook, argmin)

    s_hat = pl.pallas_call(
        _decode_body,
        grid=(B // BD,),
        in_specs=[
            pl.BlockSpec((BD, D), lambda i: (i, 0)),
            pl.BlockSpec((D, H), lambda i: (0, 0)),
            pl.BlockSpec((1, H), lambda i: (0, 0)),
            pl.BlockSpec((H, H), lambda i: (0, 0)),
            pl.BlockSpec((1, H), lambda i: (0, 0)),
            pl.BlockSpec((H, S), lambda i: (0, 0)),
            pl.BlockSpec((1, S), lambda i: (0, 0)),
        ],
        out_specs=pl.BlockSpec((BD, S), lambda i: (i, 0)),
        out_shape=jax.ShapeDtypeStruct((B, S), jnp.float32),
    )(emb, W3, b3r, W4, b4r, W5, b5r)

    return (s_hat, z_e, emb, argmin)
